# Initial kernel scaffold; baseline (speedup 1.0000x reference)
import jax
import jax.numpy as jnp
from jax.experimental import pallas as pl


def kernel(x, edge_index, edge_attr, W_e, b_e, eps, W1, b1, W2, b2):
    raise NotImplementedError("write your pallas kernel here")


# speedup vs baseline: 2.1144x; 2.1144x over previous
import functools

import jax
import jax.numpy as jnp
from jax import lax
from jax.experimental import pallas as pl
from jax.experimental.pallas import tpu as pltpu
from jax.experimental.pallas import tpu_sc as plsc

_N = 10000
_E = 320000
_D = 128
_DE = 16
_HOPS = 3

_NC = 2
_NS = 16
_NW = _NC * _NS

_B = 128
_EPT_REAL = _E // _NW
_NBLK = (_EPT_REAL + _B - 1) // _B
_NBLK = 80
_EPT = _NBLK * _B
_PAD = _EPT - _EPT_REAL
_NP = _N + _PAD
_RPT = _NP // _NS


def _sc_message_kernel():
    mesh = plsc.VectorSubcoreMesh(core_axis_name="c", subcore_axis_name="s")

    @functools.partial(
        pl.kernel,
        out_type=jax.ShapeDtypeStruct((_NC, _NP, _D), jnp.float32),
        mesh=mesh,
        scratch_types=[
            pltpu.VMEM((_B,), jnp.int32),
            pltpu.VMEM((_NBLK, _B), jnp.int32),
            pltpu.VMEM((_B, _D), jnp.float32),
            pltpu.VMEM((_B, _D), jnp.float32),
            pltpu.VMEM_SHARED((_NP, _D), jnp.float32),
            pltpu.SemaphoreType.DMA,
        ],
    )
    def sc_msg(h_hbm, e_hbm, src_hbm, dst_hbm, zeros_hbm, out_hbm,
               src_v, dst_v, g_v, e_v, agg_s, sem):
        c = lax.axis_index("c")
        s = lax.axis_index("s")
        wid = s * _NC + c

        pltpu.sync_copy(zeros_hbm, agg_s.at[pl.ds(s * _RPT, _RPT)])
        pltpu.sync_copy(dst_hbm.at[wid], dst_v)
        plsc.subcore_barrier()

        def blk(j, carry):
            pltpu.sync_copy(src_hbm.at[wid, j], src_v)
            pltpu.async_copy(h_hbm.at[src_v], g_v, sem).wait()
            pltpu.sync_copy(e_hbm.at[wid, pl.ds(j * _B, _B)], e_v)

            def row(r, rc):
                for cc in range(_D // 16):
                    sl = pl.ds(cc * 16, 16)
                    g_v[r, sl] = jnp.maximum(g_v[r, sl] + e_v[r, sl], 0.0)
                return rc

            lax.fori_loop(0, _B, row, 0)

            pltpu.sync_copy(g_v, agg_s.at[dst_v.at[j]], add=True)
            return carry

        lax.fori_loop(0, _NBLK, blk, 0)
        plsc.subcore_barrier()

        pltpu.sync_copy(agg_s.at[pl.ds(s * _RPT, _RPT)],
                        out_hbm.at[c, pl.ds(s * _RPT, _RPT)])

    return sc_msg


_sc_message = _sc_message_kernel()


def _edge_embed_body(a_ref, w_ref, b_ref, o_ref):
    o_ref[...] = jnp.maximum(
        jnp.dot(a_ref[...], w_ref[...], preferred_element_type=jnp.float32)
        + b_ref[...], 0.0)


def _edge_embed(edge_attr_pad, W_e, b_e):
    ep = edge_attr_pad.shape[0]
    blk = 2048
    grid = ep // blk
    return pl.pallas_call(
        _edge_embed_body,
        grid=(grid,),
        in_specs=[
            pl.BlockSpec((blk, _DE), lambda i: (i, 0)),
            pl.BlockSpec((_DE, _D), lambda i: (0, 0)),
            pl.BlockSpec((1, _D), lambda i: (0, 0)),
        ],
        out_specs=pl.BlockSpec((blk, _D), lambda i: (i, 0)),
        out_shape=jax.ShapeDtypeStruct((ep, _D), jnp.float32),
    )(edge_attr_pad, W_e, b_e.reshape(1, _D))


def _mlp_body(s_ref, h_ref, p_ref, w1_ref, b1_ref, w2_ref, b2_ref, o_ref):
    z = s_ref[0, 0] * h_ref[...] + p_ref[0] + p_ref[1]
    t = jnp.maximum(
        jnp.dot(z, w1_ref[...], preferred_element_type=jnp.float32)
        + b1_ref[...], 0.0)
    o_ref[...] = (
        jnp.dot(t, w2_ref[...], preferred_element_type=jnp.float32)
        + b2_ref[...])


def _mlp(scale, h, parts, W1h, b1h, W2h, b2h):
    blk = 1000
    grid = _N // blk
    return pl.pallas_call(
        _mlp_body,
        grid=(grid,),
        in_specs=[
            pl.BlockSpec(memory_space=pltpu.SMEM),
            pl.BlockSpec((blk, _D), lambda i: (i, 0)),
            pl.BlockSpec((_NC, blk, _D), lambda i: (0, i, 0)),
            pl.BlockSpec((_D, _D), lambda i: (0, 0)),
            pl.BlockSpec((1, _D), lambda i: (0, 0)),
            pl.BlockSpec((_D, _D), lambda i: (0, 0)),
            pl.BlockSpec((1, _D), lambda i: (0, 0)),
        ],
        out_specs=pl.BlockSpec((blk, _D), lambda i: (i, 0)),
        out_shape=jax.ShapeDtypeStruct((_N, _D), jnp.float32),
    )(scale, h, parts, W1h, b1h.reshape(1, _D), W2h, b2h.reshape(1, _D))


def kernel(x, edge_index, edge_attr, W_e, b_e, eps, W1, b1, W2, b2):
    src = edge_index[0]
    dst = edge_index[1]

    src3 = jnp.pad(src.reshape(_NW, _EPT_REAL), ((0, 0), (0, _PAD)))
    src3 = src3.reshape(_NW, _NBLK, _B)
    pad_dst = jnp.broadcast_to(_N + jnp.arange(_PAD, dtype=jnp.int32),
                               (_NW, _PAD))
    dst3 = jnp.concatenate([dst.reshape(_NW, _EPT_REAL), pad_dst], axis=1)
    dst3 = dst3.reshape(_NW, _NBLK, _B)
    ea = jnp.pad(edge_attr.reshape(_NW, _EPT_REAL, _DE),
                 ((0, 0), (0, _PAD), (0, 0)))
    ea = ea.reshape(_NW * _EPT, _DE)

    e = _edge_embed(ea, W_e, b_e)
    e3 = e.reshape(_NW, _EPT, _D)

    zeros = jnp.zeros((_RPT, _D), jnp.float32)

    h = x
    for hop in range(_HOPS):
        parts = _sc_message(h, e3, src3, dst3, zeros)
        scale = (1.0 + eps[hop]).reshape(1, 1)
        h = _mlp(scale, h, parts, W1[hop], b1[hop], W2[hop], b2[hop])
    return h


# speedup vs baseline: 2.6971x; 1.2756x over previous
import functools

import jax
import jax.numpy as jnp
from jax import lax
from jax.experimental import pallas as pl
from jax.experimental.pallas import tpu as pltpu
from jax.experimental.pallas import tpu_sc as plsc

_N = 10000
_E = 320000
_D = 128
_DE = 16
_HOPS = 3

_NC = 2
_NS = 16
_NW = _NC * _NS

_B = 128
_EPT_REAL = _E // _NW
_NBLK = 80
_EPT = _NBLK * _B
_PAD = _EPT - _EPT_REAL
_NP = _N + 112
_RPT = _NP // _NS


def _sc_message_kernel():
    mesh = plsc.VectorSubcoreMesh(core_axis_name="c", subcore_axis_name="s")

    @functools.partial(
        pl.kernel,
        out_type=jax.ShapeDtypeStruct((_NC, _NP, _D), jnp.float32),
        mesh=mesh,
        scratch_types=[
            pltpu.VMEM((_B,), jnp.int32),
            pltpu.VMEM((_B,), jnp.int32),
            pltpu.VMEM((_B,), jnp.int32),
            pltpu.VMEM((_B,), jnp.int32),
            pltpu.VMEM((_B, _D), jnp.float32),
            pltpu.VMEM((_B, _D), jnp.float32),
            pltpu.VMEM((_B, _D), jnp.float32),
            pltpu.VMEM_SHARED((_NP, _D), jnp.float32),
            pltpu.SemaphoreType.DMA,
            pltpu.SemaphoreType.DMA,
            pltpu.SemaphoreType.DMA,
            pltpu.SemaphoreType.DMA,
            pltpu.SemaphoreType.DMA,
        ],
    )
    def sc_msg(h_hbm, e_hbm, src_hbm, dst_hbm, zeros_hbm, out_hbm,
               s0, s1, d0, d1, g0, g1, e_v, agg_s,
               gsem0, gsem1, esem, isem0, isem1):
        c = lax.axis_index("c")
        s = lax.axis_index("s")
        wid = s * _NC + c

        sv = (s0, s1)
        dv = (d0, d1)
        gv = (g0, g1)
        gsem = (gsem0, gsem1)
        isem = (isem0, isem1)

        pltpu.sync_copy(zeros_hbm, agg_s.at[pl.ds(s * _RPT, _RPT)])
        plsc.subcore_barrier()

        def idx_start(j, slot):
            pltpu.async_copy(src_hbm.at[wid, j], sv[slot], isem[slot])
            pltpu.async_copy(dst_hbm.at[wid, j], dv[slot], isem[slot])

        def idx_wait(j, slot):
            pltpu.make_async_copy(src_hbm.at[wid, j], sv[slot],
                                  isem[slot]).wait()
            pltpu.make_async_copy(dst_hbm.at[wid, j], dv[slot],
                                  isem[slot]).wait()

        def gather_start(slot):
            pltpu.async_copy(h_hbm.at[sv[slot]], gv[slot], gsem[slot])

        def gather_wait(slot):
            pltpu.make_async_copy(h_hbm.at[sv[slot]], gv[slot],
                                  gsem[slot]).wait()

        def e_start(j):
            pltpu.async_copy(e_hbm.at[wid, pl.ds(j * _B, _B)], e_v, esem)

        def e_wait(j):
            pltpu.make_async_copy(e_hbm.at[wid, pl.ds(j * _B, _B)], e_v,
                                  esem).wait()

        def process(j, slot):
            other = 1 - slot
            def launch_next():
                idx_wait(j + 1, other)
                gather_start(other)
            pl.when(j + 1 < _NBLK)(launch_next)

            gather_wait(slot)
            e_wait(j)
            g = gv[slot]

            def row(r, rc):
                for cc in range(_D // 16):
                    sl = pl.ds(cc * 16, 16)
                    g[r, sl] = jnp.maximum(g[r, sl] + e_v[r, sl], 0.0)
                return rc

            lax.fori_loop(0, _B, row, 0)
            pl.when(j + 1 < _NBLK)(lambda: e_start(j + 1))
            pltpu.sync_copy(g, agg_s.at[dv[slot]], add=True)
            pl.when(j + 2 < _NBLK)(lambda: idx_start(j + 2, slot))

        pltpu.sync_copy(src_hbm.at[wid, 0], s0)
        pltpu.sync_copy(dst_hbm.at[wid, 0], d0)
        gather_start(0)
        e_start(0)
        idx_start(1, 1)

        def pair(i, carry):
            process(2 * i, 0)
            process(2 * i + 1, 1)
            return carry

        lax.fori_loop(0, _NBLK // 2, pair, 0)
        plsc.subcore_barrier()

        pltpu.sync_copy(agg_s.at[pl.ds(s * _RPT, _RPT)],
                        out_hbm.at[c, pl.ds(s * _RPT, _RPT)])

    return sc_msg


_sc_message = _sc_message_kernel()


def _edge_embed_body(a_ref, w_ref, b_ref, o_ref):
    o_ref[...] = jnp.maximum(
        jnp.dot(a_ref[...], w_ref[...], preferred_element_type=jnp.float32)
        + b_ref[...], 0.0)


def _edge_embed(edge_attr_pad, W_e, b_e):
    ep = edge_attr_pad.shape[0]
    blk = 2048
    grid = ep // blk
    return pl.pallas_call(
        _edge_embed_body,
        grid=(grid,),
        in_specs=[
            pl.BlockSpec((blk, _DE), lambda i: (i, 0)),
            pl.BlockSpec((_DE, _D), lambda i: (0, 0)),
            pl.BlockSpec((1, _D), lambda i: (0, 0)),
        ],
        out_specs=pl.BlockSpec((blk, _D), lambda i: (i, 0)),
        out_shape=jax.ShapeDtypeStruct((ep, _D), jnp.float32),
    )(edge_attr_pad, W_e, b_e.reshape(1, _D))


def _mlp_body(s_ref, h_ref, p_ref, w1_ref, b1_ref, w2_ref, b2_ref, o_ref):
    z = s_ref[0, 0] * h_ref[...] + p_ref[0] + p_ref[1]
    t = jnp.maximum(
        jnp.dot(z, w1_ref[...], preferred_element_type=jnp.float32)
        + b1_ref[...], 0.0)
    o_ref[...] = (
        jnp.dot(t, w2_ref[...], preferred_element_type=jnp.float32)
        + b2_ref[...])


def _mlp(scale, h, parts, W1h, b1h, W2h, b2h):
    blk = 1000
    grid = _N // blk
    return pl.pallas_call(
        _mlp_body,
        grid=(grid,),
        in_specs=[
            pl.BlockSpec(memory_space=pltpu.SMEM),
            pl.BlockSpec((blk, _D), lambda i: (i, 0)),
            pl.BlockSpec((_NC, blk, _D), lambda i: (0, i, 0)),
            pl.BlockSpec((_D, _D), lambda i: (0, 0)),
            pl.BlockSpec((1, _D), lambda i: (0, 0)),
            pl.BlockSpec((_D, _D), lambda i: (0, 0)),
            pl.BlockSpec((1, _D), lambda i: (0, 0)),
        ],
        out_specs=pl.BlockSpec((blk, _D), lambda i: (i, 0)),
        out_shape=jax.ShapeDtypeStruct((_N, _D), jnp.float32),
    )(scale, h, parts, W1h, b1h.reshape(1, _D), W2h, b2h.reshape(1, _D))


def kernel(x, edge_index, edge_attr, W_e, b_e, eps, W1, b1, W2, b2):
    src = edge_index[0]
    dst = edge_index[1]

    src3 = jnp.pad(src.reshape(_NW, _EPT_REAL), ((0, 0), (0, _PAD)))
    src3 = src3.reshape(_NW, _NBLK, _B)
    pad_dst = jnp.broadcast_to(
        _N + (jnp.arange(_PAD, dtype=jnp.int32) % (_NP - _N)), (_NW, _PAD))
    dst3 = jnp.concatenate([dst.reshape(_NW, _EPT_REAL), pad_dst], axis=1)
    dst3 = dst3.reshape(_NW, _NBLK, _B)
    ea = jnp.pad(edge_attr.reshape(_NW, _EPT_REAL, _DE),
                 ((0, 0), (0, _PAD), (0, 0)))
    ea = ea.reshape(_NW * _EPT, _DE)

    e = _edge_embed(ea, W_e, b_e)
    e3 = e.reshape(_NW, _EPT, _D)

    zeros = jnp.zeros((_RPT, _D), jnp.float32)

    h = x
    for hop in range(_HOPS):
        parts = _sc_message(h, e3, src3, dst3, zeros)
        scale = (1.0 + eps[hop]).reshape(1, 1)
        h = _mlp(scale, h, parts, W1[hop], b1[hop], W2[hop], b2[hop])
    return h
